# Initial kernel scaffold; baseline (speedup 1.0000x reference)
import jax
import jax.numpy as jnp
from jax.experimental import pallas as pl


def kernel(inputs, adj_mx, forward_index, W_gate0, b_gate0, W_cand0, b_cand0, W_gate1, b_gate1, W_cand1, b_cand1):
    raise NotImplementedError("write your pallas kernel here")


# speedup vs baseline: 8.2490x; 8.2490x over previous
import jax
import jax.numpy as jnp
from jax.experimental import pallas as pl

N = 1024
U = 64
B = 8
IN = 2
NM = 5


def _fused(adj_ref, adjT_ref, x0_ref, w0c_ref, w0u_ref, w1c_ref, w1u_ref,
           bc0_ref, bu0_ref, bc1_ref, bu1_ref, nh0_ref, nh1_ref):
    adj = adj_ref[...]
    adjT = adjT_ref[...]

    d_row = jnp.sum(adj, axis=1, keepdims=True)
    d_col = jnp.sum(adjT, axis=1, keepdims=True)
    dinv0 = jnp.where(d_row > 0.0, 1.0 / d_row, 0.0)
    dinv1 = jnp.where(d_col > 0.0, 1.0 / d_col, 0.0)

    def diffuse(x):
        z1 = jnp.dot(adjT, dinv0 * x, preferred_element_type=jnp.float32)
        z2 = 2.0 * jnp.dot(adjT, dinv0 * z1, preferred_element_type=jnp.float32) - x
        z3 = jnp.dot(adj, dinv1 * x, preferred_element_type=jnp.float32)
        z4 = 2.0 * jnp.dot(adj, dinv1 * z3, preferred_element_type=jnp.float32) - x
        return [x, z1, z2, z3, z4]

    xs0 = diffuse(x0_ref[...])
    xb0 = jnp.concatenate(xs0, axis=1)
    cc0 = jnp.dot(xb0, w0c_ref[...], preferred_element_type=jnp.float32) + bc0_ref[...]
    uu0 = jnp.dot(xb0, w0u_ref[...], preferred_element_type=jnp.float32) + bu0_ref[...]
    h0 = (1.0 - jax.nn.sigmoid(uu0)) * jnp.tanh(cc0)

    xs1 = diffuse(h0)
    rows = []
    for b in range(B):
        sl = slice(b * U, (b + 1) * U)
        rows.append(jnp.concatenate([z[:, sl] for z in xs1], axis=1))
    xb1 = jnp.concatenate(rows, axis=0)
    cc1 = jnp.dot(xb1, w1c_ref[...], preferred_element_type=jnp.float32) + bc1_ref[...]
    uu1 = jnp.dot(xb1, w1u_ref[...], preferred_element_type=jnp.float32) + bu1_ref[...]

    nh0_ref[...] = xb1[:, :U]
    nh1_ref[...] = (1.0 - jax.nn.sigmoid(uu1)) * jnp.tanh(cc1)


def kernel(inputs, adj_mx, forward_index, W_gate0, b_gate0, W_cand0, b_cand0,
           W_gate1, b_gate1, W_cand1, b_cand1):
    f32 = jnp.float32
    adj = adj_mx.astype(f32)
    adjT = adj.T

    x0 = inputs.reshape(B, N, IN).transpose(1, 0, 2).reshape(N, B * IN)

    w0c_small = W_cand0[: IN * NM].reshape(IN, NM, U).transpose(1, 0, 2)
    w0u_small = W_gate0[: IN * NM, U: 2 * U].reshape(IN, NM, U).transpose(1, 0, 2)
    eye_b = jnp.eye(B, dtype=f32)[None, :, None, :, None]

    def blockdiag(small):
        return (eye_b * small[:, None, :, None, :]).reshape(NM * B * IN, B * U)

    w0c = blockdiag(w0c_small)
    w0u = blockdiag(w0u_small)

    w1c = W_cand1[: U * NM].reshape(U, NM, U).transpose(1, 0, 2).reshape(NM * U, U)
    w1u = W_gate1[: U * NM, U: 2 * U].reshape(U, NM, U).transpose(1, 0, 2).reshape(NM * U, U)

    bc0 = jnp.tile(b_cand0, B).reshape(1, B * U)
    bu0 = jnp.tile(b_gate0[U: 2 * U], B).reshape(1, B * U)
    bc1 = b_cand1.reshape(1, U)
    bu1 = b_gate1[U: 2 * U].reshape(1, U)

    nh0_bm, nh1_bm = pl.pallas_call(
        _fused,
        out_shape=[
            jax.ShapeDtypeStruct((B * N, U), f32),
            jax.ShapeDtypeStruct((B * N, U), f32),
        ],
    )(adj, adjT, x0, w0c, w0u, w1c, w1u, bc0, bu0, bc1, bu1)

    nh0 = nh0_bm.reshape(B, N * U)
    nh1 = nh1_bm.reshape(B, N * U)
    hidden = jnp.stack([nh0, nh1], axis=0)
    return (nh1, hidden)


# speedup vs baseline: 9.2893x; 1.1261x over previous
import jax
import jax.numpy as jnp
from jax import lax
from jax.experimental import pallas as pl

N = 1024
U = 64
B = 8
IN = 2
NM = 5

_TN = (((0,), (0,)), ((), ()))


def _fused(adj_ref, x0_ref, ones_ref, w0_ref, w1_ref, b0_ref, b1_ref,
           hid_ref, nh1_ref):
    f32 = jnp.float32
    adj = adj_ref[...]

    ones = ones_ref[...]
    d_row = jnp.dot(adj, ones, preferred_element_type=f32)
    d_col = lax.dot_general(adj, ones, _TN, preferred_element_type=f32)
    dinv0 = jnp.where(d_row > 0.0, 1.0 / d_row, 0.0)
    dinv1 = jnp.where(d_col > 0.0, 1.0 / d_col, 0.0)

    def diffuse(x):
        z1 = lax.dot_general(adj, dinv0 * x, _TN, preferred_element_type=f32)
        z2 = 2.0 * lax.dot_general(adj, dinv0 * z1, _TN, preferred_element_type=f32) - x
        z3 = jnp.dot(adj, dinv1 * x, preferred_element_type=f32)
        z4 = 2.0 * jnp.dot(adj, dinv1 * z3, preferred_element_type=f32) - x
        return [x, z1, z2, z3, z4]

    xs0 = diffuse(x0_ref[...])
    xb0 = jnp.concatenate(xs0, axis=1)
    cu0 = jnp.dot(xb0, w0_ref[...], preferred_element_type=f32) + b0_ref[...]
    cc0, uu0 = cu0[:, : B * U], cu0[:, B * U:]
    h0 = (1.0 - jax.nn.sigmoid(uu0)) * jnp.tanh(cc0)

    xs1 = diffuse(h0)
    rows = []
    for b in range(B):
        sl = slice(b * U, (b + 1) * U)
        rows.append(jnp.concatenate([z[:, sl] for z in xs1], axis=1))
    xb1 = jnp.concatenate(rows, axis=0)
    cu1 = jnp.dot(xb1, w1_ref[...], preferred_element_type=f32) + b1_ref[...]
    nh1 = (1.0 - jax.nn.sigmoid(cu1[:, U:])) * jnp.tanh(cu1[:, :U])

    hid_ref[: B * N, :] = xb1[:, :U]
    hid_ref[B * N:, :] = nh1
    nh1_ref[...] = nh1


def kernel(inputs, adj_mx, forward_index, W_gate0, b_gate0, W_cand0, b_cand0,
           W_gate1, b_gate1, W_cand1, b_cand1):
    f32 = jnp.float32
    adj = adj_mx.astype(f32)

    x0 = inputs.reshape(B, N, IN).transpose(1, 0, 2).reshape(N, B * IN)

    w0c_small = W_cand0[: IN * NM].reshape(IN, NM, U).transpose(1, 0, 2)
    w0u_small = W_gate0[: IN * NM, U: 2 * U].reshape(IN, NM, U).transpose(1, 0, 2)
    eye_b = jnp.eye(B, dtype=f32)[None, :, None, :, None]

    def blockdiag(small):
        return (eye_b * small[:, None, :, None, :]).reshape(NM * B * IN, B * U)

    w0 = jnp.concatenate([blockdiag(w0c_small), blockdiag(w0u_small)], axis=1)

    w1c = W_cand1[: U * NM].reshape(U, NM, U).transpose(1, 0, 2).reshape(NM * U, U)
    w1u = W_gate1[: U * NM, U: 2 * U].reshape(U, NM, U).transpose(1, 0, 2).reshape(NM * U, U)
    w1 = jnp.concatenate([w1c, w1u], axis=1)

    b0 = jnp.concatenate([jnp.tile(b_cand0, B), jnp.tile(b_gate0[U: 2 * U], B)]).reshape(1, 2 * B * U)
    b1 = jnp.concatenate([b_cand1, b_gate1[U: 2 * U]]).reshape(1, 2 * U)
    ones = jnp.ones((N, 1), f32)

    hid_bm, nh1_bm = pl.pallas_call(
        _fused,
        out_shape=[
            jax.ShapeDtypeStruct((2 * B * N, U), f32),
            jax.ShapeDtypeStruct((B * N, U), f32),
        ],
    )(adj, x0, ones, w0, w1, b0, b1)

    nh1 = nh1_bm.reshape(B, N * U)
    hidden = hid_bm.reshape(2, B, N * U)
    return (nh1, hidden)
